# baseline (speedup 1.0000x reference)
import functools
from math import sqrt

import jax
import jax.numpy as jnp
from jax import lax
from jax.experimental import pallas as pl
from jax.experimental.pallas import tpu as pltpu
from jax.experimental.pallas import tpu_sc as plsc

_VOCAB = 1000000
_D = 64
_ROWS = 4096
_COLS = 200
_B = _ROWS * _COLS
_NC = 2
_NS = 16
_NW = _NC * _NS
_PER_W = _B // _NW
_CHUNK = 128
_NCHUNK = _PER_W // _CHUNK
_SCALE = sqrt(_D)


def _body(x_hbm, tab_hbm, out_hbm, idx_v, rows_v, sem):
    cid = lax.axis_index("c")
    sid = lax.axis_index("s")
    wid = sid * _NC + cid
    pltpu.sync_copy(x_hbm.at[wid], idx_v)

    @pl.loop(0, _NCHUNK)
    def _chunk(j):
        pltpu.async_copy(tab_hbm.at[idx_v.at[j]], rows_v, sem).wait()

        @pl.loop(0, _CHUNK)
        def _row(r):
            for c in range(_D // 16):
                sl = pl.ds(c * 16, 16)
                rows_v[r, sl] = rows_v[r, sl] * _SCALE

        pltpu.sync_copy(rows_v, out_hbm.at[wid, j])


@jax.jit
def _embed(x_flat, table):
    k = pl.kernel(
        _body,
        out_type=jax.ShapeDtypeStruct((_NW, _NCHUNK, _CHUNK, _D), jnp.float32),
        mesh=plsc.VectorSubcoreMesh(core_axis_name="c", subcore_axis_name="s"),
        scratch_types=[
            pltpu.VMEM((_NCHUNK, _CHUNK), jnp.int32),
            pltpu.VMEM((_CHUNK, _D), jnp.float32),
            pltpu.SemaphoreType.DMA,
        ],
        compiler_params=pltpu.CompilerParams(use_tc_tiling_on_sc=False),
    )
    return k(x_flat, table)


def kernel(x, embedding_weight):
    x_flat = x.astype(jnp.int32).reshape(_NW, _NCHUNK, _CHUNK)
    out = _embed(x_flat, embedding_weight)
    return out.reshape(_ROWS, _COLS, _D)


# speedup vs baseline: 1.0978x; 1.0978x over previous
import functools
from math import sqrt

import jax
import jax.numpy as jnp
from jax import lax
from jax.experimental import pallas as pl
from jax.experimental.pallas import tpu as pltpu
from jax.experimental.pallas import tpu_sc as plsc

_VOCAB = 1000000
_D = 64
_ROWS = 4096
_COLS = 200
_B = _ROWS * _COLS
_NC = 2
_NS = 16
_NW = _NC * _NS
_PER_W = _B // _NW
_CHUNK = 128
_NCHUNK = _PER_W // _CHUNK
_NBUF = 4
_SCALE = sqrt(_D)


def _scale_chunk(rows_in, rows_out, b):
    @pl.loop(0, _CHUNK, unroll=4)
    def _row(r):
        for c in range(_D // 16):
            sl = pl.ds(c * 16, 16)
            rows_out[b, r, sl] = rows_in[b, r, sl] * _SCALE


def _body(x_hbm, tab_hbm, out_hbm, idx_v, rows_in, rows_out, gsem, wsem):
    cid = lax.axis_index("c")
    sid = lax.axis_index("s")
    wid = sid * _NC + cid
    pltpu.sync_copy(x_hbm.at[wid], idx_v)

    def fire_gather(k, b):
        pltpu.async_copy(tab_hbm.at[idx_v.at[k]], rows_in.at[b], gsem)

    def drain_gather(b):
        pltpu.make_async_copy(tab_hbm.at[idx_v.at[0]], rows_in.at[b], gsem).wait()

    def fire_write(k, b):
        pltpu.async_copy(rows_out.at[b], out_hbm.at[wid, k], wsem)

    def drain_write(b):
        pltpu.make_async_copy(rows_out.at[b], out_hbm.at[wid, 0], wsem).wait()

    for b in range(_NBUF):
        fire_gather(b, b)

    @pl.loop(0, _NCHUNK - _NBUF, step=_NBUF)
    def _group(j):
        for b in range(_NBUF):
            k = j + b
            drain_gather(b)
            _scale_chunk(rows_in, rows_out, b)
            fire_gather(k + _NBUF, b)

            @pl.when(j > 0)
            def _():
                drain_write(b)

            fire_write(k, b)

    for b in range(_NBUF):
        k = _NCHUNK - _NBUF + b
        drain_gather(b)
        _scale_chunk(rows_in, rows_out, b)
        drain_write(b)
        fire_write(k, b)
    for b in range(_NBUF):
        drain_write(b)


@jax.jit
def _embed(x_flat, table):
    k = pl.kernel(
        _body,
        out_type=jax.ShapeDtypeStruct((_NW, _NCHUNK, _CHUNK, _D), jnp.float32),
        mesh=plsc.VectorSubcoreMesh(core_axis_name="c", subcore_axis_name="s"),
        scratch_types=[
            pltpu.VMEM((_NCHUNK, _CHUNK), jnp.int32),
            pltpu.VMEM((_NBUF, _CHUNK, _D), jnp.float32),
            pltpu.VMEM((_NBUF, _CHUNK, _D), jnp.float32),
            pltpu.SemaphoreType.DMA,
            pltpu.SemaphoreType.DMA,
        ],
        compiler_params=pltpu.CompilerParams(use_tc_tiling_on_sc=False),
    )
    return k(x_flat, table)


def kernel(x, embedding_weight):
    x_flat = x.astype(jnp.int32).reshape(_NW, _NCHUNK, _CHUNK)
    out = _embed(x_flat, embedding_weight)
    return out.reshape(_ROWS, _COLS, _D)
